# baseline (speedup 1.0000x reference)
import functools

import jax
import jax.numpy as jnp
from jax import lax
from jax.experimental import pallas as pl
from jax.experimental.pallas import tpu as pltpu
from jax.experimental.pallas import tpu_sc as plsc

N = 10000
E = 320000
D = 128
PW = 144
NC, NS = 2, 16
EPC = E // NC
EPT = EPC // NS
K = 80
H0, H1 = 48, 32
NB = EPT // K
TB = E // K
NP = 10240
RPT = NP // NS
ZR = 40
WR = NP // D
BR = 1000


def _proj_body(x_ref, emb_ref, w_ref, vi_ref, vj_ref, vei_ref, vej_ref,
               xl_ref, aij_ref):
    xl = jnp.dot(x_ref[...], w_ref[...], preferred_element_type=jnp.float32)
    xl_ref[...] = xl
    emb = emb_ref[...]
    ai = jnp.sum(xl * vi_ref[...], axis=1) + jnp.sum(emb * vei_ref[...], axis=1)
    aj = jnp.sum(xl * vj_ref[...], axis=1) + jnp.sum(emb * vej_ref[...], axis=1)
    aij_ref[0, :] = ai
    aij_ref[1, :] = aj


_proj = pl.pallas_call(
    _proj_body,
    out_shape=[
        jax.ShapeDtypeStruct((N, D), jnp.float32),
        jax.ShapeDtypeStruct((2, N), jnp.float32),
    ],
)


@functools.partial(
    pl.kernel,
    out_type=[
        jax.ShapeDtypeStruct((NC, NP, D), jnp.float32),
        jax.ShapeDtypeStruct((NC, WR, D), jnp.float32),
    ],
    mesh=plsc.VectorSubcoreMesh(core_axis_name="c", subcore_axis_name="s"),
    scratch_types=[
        pltpu.VMEM((N,), jnp.float32),
        pltpu.VMEM((N,), jnp.float32),
        pltpu.VMEM((2, K), jnp.int32),
        pltpu.VMEM((2, K), jnp.int32),
        pltpu.VMEM((K, D), jnp.float32),
        pltpu.VMEM((K // 16, 16), jnp.int32),
        pltpu.VMEM((K // 16, 16), jnp.float32),
        pltpu.VMEM((WR, D), jnp.float32),
        pltpu.VMEM((WR,), jnp.int32),
        pltpu.SemaphoreType.DMA,
        pltpu.SemaphoreType.DMA,
        pltpu.SemaphoreType.DMA,
        pltpu.SemaphoreType.DMA,
        pltpu.SemaphoreType.DMA,
        pltpu.SemaphoreType.DMA,
        pltpu.VMEM_SHARED((NP, D), jnp.float32),
        pltpu.VMEM_SHARED((WR, D), jnp.float32),
    ],
    compiler_params=pltpu.CompilerParams(needs_layout_passes=False),
)
def _edge_kernel(eib_hbm, xl_hbm, aij_hbm, parts_hbm, partsw_hbm,
                 ai_v, aj_v, ib0_v, ib1_v, rows_v,
                 ktmp_v, ctmp_v, accw_v, wid_v, semA, semB, semS, semS2,
                 semI0, semI1, accf_sh, accw_sh):
    c = lax.axis_index("c")
    s = lax.axis_index("s")
    base_b = c * (EPC // K) + s * NB

    pltpu.sync_copy(aij_hbm.at[0], ai_v)
    pltpu.sync_copy(aij_hbm.at[1], aj_v)

    zero16 = jnp.zeros((16,), jnp.float32)
    lanes = lax.iota(jnp.int32, 16)
    def zw(r, carry):
        for cc in range(D // 16):
            accw_v[r, pl.ds(cc * 16, 16)] = zero16
        return carry

    lax.fori_loop(0, WR, zw, 0)

    def zi(i, carry):
        wid_v[pl.ds(i * 16, 16)] = lanes + i * 16
        return carry

    lax.fori_loop(0, WR // 16, zi, 0)

    for z in range(RPT // WR):
        pltpu.sync_copy(accw_v, accf_sh.at[pl.ds(s * RPT + z * WR, WR)])

    @pl.when(s == 0)
    def _():
        pltpu.sync_copy(accw_v, accw_sh)

    plsc.subcore_barrier()

    def zr1(r, carry):
        for cc in range(D // 16):
            rows_v[H0 + r, pl.ds(cc * 16, 16)] = zero16
        return carry

    lax.fori_loop(0, H1, zr1, 0)
    pltpu.async_copy(rows_v.at[pl.ds(H0, H1)],
                     accf_sh.at[wid_v.at[pl.ds(0, H1)]], semS2, add=True)

    ibufs = (ib0_v, ib1_v)
    isems = (semI0, semI1)

    def _drain_s2(b):
        pltpu.make_async_copy(
            rows_v.at[pl.ds(H0, H1)],
            accf_sh.at[ibufs[b ^ 1].at[1, pl.ds(H0, H1)]], semS2).wait()

    def load_idx(g, b):
        pltpu.async_copy(eib_hbm.at[base_b + g], ibufs[b], isems[b])

    def load_idx_wait(g, b):
        pltpu.make_async_copy(eib_hbm.at[base_b + g], ibufs[b],
                              isems[b]).wait()

    def gather(b, off, n, sem):
        pltpu.async_copy(xl_hbm.at[ibufs[b].at[0, pl.ds(off, n)]],
                         rows_v.at[pl.ds(off, n)], sem)

    def gather_wait(b, off, n, sem):
        pltpu.make_async_copy(xl_hbm.at[ibufs[b].at[0, pl.ds(off, n)]],
                              rows_v.at[pl.ds(off, n)], sem).wait()

    def compute(b, off, n):
        ib = ibufs[b]

        @plsc.parallel_loop(off // 16, off // 16 + n // 16,
                            unroll=n // 16)
        def chunk(j):
            j0 = j * 16
            sv = ib[0, pl.ds(j0, 16)]
            dv = ib[1, pl.ds(j0, 16)]
            av = plsc.load_gather(ai_v, [dv]) + plsc.load_gather(aj_v, [sv])
            av = jnp.where(av > 0, av, av * 0.2)
            wv = jnp.where(sv == dv, 0.0, jnp.exp(av))
            for l in range(16):
                wsp = jnp.take_along_axis(
                    wv, jnp.full((16,), l, jnp.int32), axis=0)
                rr = j0 + l
                for cc in range(D // 16):
                    rows_v[rr, pl.ds(cc * 16, 16)] = (
                        rows_v[rr, pl.ds(cc * 16, 16)] * wsp)
            ks, vs = plsc.sort_key_val(dv, wv)
            cs = plsc.cumsum(vs)
            jsplat = jnp.full((16,), j, jnp.int32)
            ktmp_v[j, pl.ds(0, 16)] = ks
            ctmp_v[j, pl.ds(0, 16)] = cs
            kprev = plsc.load_gather(ktmp_v, [jsplat, jnp.maximum(lanes - 1, 0)])
            knext = plsc.load_gather(ktmp_v, [jsplat, jnp.minimum(lanes + 1, 15)])
            start = jnp.logical_or(lanes == 0, ks != kprev)
            end = jnp.logical_or(lanes == 15, ks != knext)
            st = plsc.cummax(jnp.where(start, lanes, 0))
            cprev = plsc.load_gather(ctmp_v, [jsplat, jnp.maximum(st - 1, 0)])
            cprev = jnp.where(st == 0, 0.0, cprev)
            plsc.addupdate_scatter(
                accw_v, [jnp.right_shift(ks, 7), jnp.bitwise_and(ks, 127)],
                cs - cprev, mask=end)

    def scatter(b, off, n, sem):
        pltpu.async_copy(rows_v.at[pl.ds(off, n)],
                         accf_sh.at[ibufs[b].at[1, pl.ds(off, n)]],
                         sem, add=True)

    def scatter_wait(b, off, n, sem):
        pltpu.make_async_copy(rows_v.at[pl.ds(off, n)],
                              accf_sh.at[ibufs[b].at[1, pl.ds(off, n)]],
                              sem).wait()

    def run_batch2(g, b, prefetch):
        _drain_s2(b)
        if prefetch:
            load_idx(g + 1, b ^ 1)
        gather(b, H0, H1, semB)
        gather_wait(b, 0, H0, semA)
        compute(b, 0, H0)
        scatter(b, 0, H0, semS)
        gather_wait(b, H0, H1, semB)
        compute(b, H0, H1)
        scatter_wait(b, 0, H0, semS)
        scatter(b, H0, H1, semS2)
        if prefetch:
            load_idx_wait(g + 1, b ^ 1)
            gather(b ^ 1, 0, H0, semA)

    load_idx(0, 0)
    load_idx_wait(0, 0)
    gather(0, 0, H0, semA)

    def pairbody(i, carry):
        run_batch2(2 * i, 0, True)
        run_batch2(2 * i + 1, 1, True)
        return carry

    lax.fori_loop(0, NB // 2, pairbody, 0)
    run_batch2(NB - 1, 0, False)
    _drain_s2(0)

    pltpu.sync_copy(accw_v, accw_sh.at[wid_v], add=True)
    plsc.subcore_barrier()

    pltpu.sync_copy(accf_sh.at[pl.ds(s * RPT, RPT)],
                    parts_hbm.at[c, pl.ds(s * RPT, RPT)])

    @pl.when(s < WR // 8)
    def _():
        pltpu.sync_copy(accw_sh.at[pl.ds(s * 8, 8)],
                        partsw_hbm.at[c, pl.ds(s * 8, 8)])


def _comb_bn_body(parts_ref, pw_ref, xl_ref, sa_ref, bias_ref,
                  gamma_ref, beta_ref, out_ref):
    accf = parts_ref[0, :N] + parts_ref[1, :N]
    accw = pw_ref[0, :N] + pw_ref[1, :N]
    als = sa_ref[...]
    als = jnp.where(als > 0, als, als * 0.2)
    ws = jnp.exp(als)
    num = accf + ws * xl_ref[...]
    den = accw + ws
    o = num / den + bias_ref[...]
    mu = jnp.mean(o, axis=0, keepdims=True)
    xc = o - mu
    var = jnp.mean(xc * xc, axis=0, keepdims=True)
    y = xc * lax.rsqrt(var + 1e-5) * gamma_ref[...] + beta_ref[...]
    out_ref[...] = jnp.maximum(y, 0.0)


_comb_bn = pl.pallas_call(
    _comb_bn_body,
    out_shape=jax.ShapeDtypeStruct((N, D), jnp.float32),
)


def kernel(x, edge_index, embedding, W, att_i, att_j, att_em_i, att_em_j,
           bias, gamma, beta):
    vi = att_i.reshape(1, D)
    vj = att_j.reshape(1, D)
    vei = att_em_i.reshape(1, D)
    vej = att_em_j.reshape(1, D)
    xl, aij = _proj(x, embedding, W, vi, vj, vei, vej)
    eib = edge_index.reshape(2, TB, K).transpose(1, 0, 2)
    parts, partsw = _edge_kernel(eib, xl, aij)
    pw = partsw.reshape(NC, NP, 1)
    sa_col = (aij[0] + aij[1])[:, None]
    return _comb_bn(parts, pw, xl, sa_col, bias.reshape(1, D),
                    gamma.reshape(1, D), beta.reshape(1, D))


# speedup vs baseline: 1.2009x; 1.2009x over previous
import functools

import jax
import jax.numpy as jnp
from jax import lax
from jax.experimental import pallas as pl
from jax.experimental.pallas import tpu as pltpu
from jax.experimental.pallas import tpu_sc as plsc

N = 10000
E = 320000
D = 128
PW = 144
NC, NS = 2, 16
EPC = E // NC
EPT = EPC // NS
K = 80
H0, H1 = 48, 32
NB = EPT // K
TB = E // K
NP = 10240
RPT = NP // NS
ZR = 40
WR = NP // D
BR = 1000


def _proj_body(x_ref, emb_ref, w_ref, vi_ref, vj_ref, vei_ref, vej_ref,
               xl_ref, aij_ref):
    xl = jnp.dot(x_ref[...], w_ref[...], preferred_element_type=jnp.float32)
    xl_ref[...] = xl
    emb = emb_ref[...]
    ai = jnp.sum(xl * vi_ref[...], axis=1) + jnp.sum(emb * vei_ref[...], axis=1)
    aj = jnp.sum(xl * vj_ref[...], axis=1) + jnp.sum(emb * vej_ref[...], axis=1)
    aij_ref[0, :] = ai
    aij_ref[1, :] = aj


_proj = pl.pallas_call(
    _proj_body,
    out_shape=[
        jax.ShapeDtypeStruct((N, D), jnp.float32),
        jax.ShapeDtypeStruct((2, N), jnp.float32),
    ],
)


@functools.partial(
    pl.kernel,
    out_type=[
        jax.ShapeDtypeStruct((NC, NP, D), jnp.float32),
        jax.ShapeDtypeStruct((NC, WR, D), jnp.float32),
    ],
    mesh=plsc.VectorSubcoreMesh(core_axis_name="c", subcore_axis_name="s"),
    scratch_types=[
        pltpu.VMEM((N,), jnp.float32),
        pltpu.VMEM((N,), jnp.float32),
        pltpu.VMEM((2, K), jnp.int32),
        pltpu.VMEM((2, K), jnp.int32),
        pltpu.VMEM((K, D), jnp.float32),
        pltpu.VMEM((K // 16, 16), jnp.int32),
        pltpu.VMEM((K // 16, 16), jnp.float32),
        pltpu.VMEM((WR, D), jnp.float32),
        pltpu.VMEM((WR,), jnp.int32),
        pltpu.SemaphoreType.DMA,
        pltpu.SemaphoreType.DMA,
        pltpu.SemaphoreType.DMA,
        pltpu.SemaphoreType.DMA,
        pltpu.SemaphoreType.DMA,
        pltpu.SemaphoreType.DMA,
        pltpu.VMEM_SHARED((NP, D), jnp.float32),
        pltpu.VMEM_SHARED((WR, D), jnp.float32),
    ],
    compiler_params=pltpu.CompilerParams(needs_layout_passes=False),
)
def _edge_kernel(eib_hbm, xl_hbm, aij_hbm, parts_hbm, partsw_hbm,
                 ai_v, aj_v, ib0_v, ib1_v, rows_v,
                 ktmp_v, ctmp_v, accw_v, wid_v, semA, semB, semS, semS2,
                 semI0, semI1, accf_sh, accw_sh):
    c = lax.axis_index("c")
    s = lax.axis_index("s")
    base_b = c * (EPC // K) + s * NB

    pltpu.sync_copy(aij_hbm.at[0], ai_v)
    pltpu.sync_copy(aij_hbm.at[1], aj_v)

    zero16 = jnp.zeros((16,), jnp.float32)
    lanes = lax.iota(jnp.int32, 16)
    def zw(r, carry):
        for cc in range(D // 16):
            accw_v[r, pl.ds(cc * 16, 16)] = zero16
        return carry

    lax.fori_loop(0, WR, zw, 0)

    def zi(i, carry):
        wid_v[pl.ds(i * 16, 16)] = lanes + i * 16
        return carry

    lax.fori_loop(0, WR // 16, zi, 0)

    for z in range(RPT // WR):
        pltpu.sync_copy(accw_v, accf_sh.at[pl.ds(s * RPT + z * WR, WR)])

    @pl.when(s == 0)
    def _():
        pltpu.sync_copy(accw_v, accw_sh)

    plsc.subcore_barrier()

    def zr1(r, carry):
        for cc in range(D // 16):
            rows_v[H0 + r, pl.ds(cc * 16, 16)] = zero16
        return carry

    lax.fori_loop(0, H1, zr1, 0)
    pltpu.async_copy(rows_v.at[pl.ds(H0, H1)],
                     accf_sh.at[wid_v.at[pl.ds(0, H1)]], semS2, add=True)

    ibufs = (ib0_v, ib1_v)
    isems = (semI0, semI1)

    def _drain_s2(b):
        pltpu.make_async_copy(
            rows_v.at[pl.ds(H0, H1)],
            accf_sh.at[ibufs[b ^ 1].at[1, pl.ds(H0, H1)]], semS2).wait()

    def load_idx(g, b):
        pltpu.async_copy(eib_hbm.at[base_b + g], ibufs[b], isems[b])

    def load_idx_wait(g, b):
        pltpu.make_async_copy(eib_hbm.at[base_b + g], ibufs[b],
                              isems[b]).wait()

    def gather(b, off, n, sem):
        pltpu.async_copy(xl_hbm.at[ibufs[b].at[0, pl.ds(off, n)]],
                         rows_v.at[pl.ds(off, n)], sem)

    def gather_wait(b, off, n, sem):
        pltpu.make_async_copy(xl_hbm.at[ibufs[b].at[0, pl.ds(off, n)]],
                              rows_v.at[pl.ds(off, n)], sem).wait()

    def compute(b, off, n):
        ib = ibufs[b]

        @plsc.parallel_loop(off // 16, off // 16 + n // 16)
        def chunk(j):
            j0 = j * 16
            sv = ib[0, pl.ds(j0, 16)]
            dv = ib[1, pl.ds(j0, 16)]
            av = plsc.load_gather(ai_v, [dv]) + plsc.load_gather(aj_v, [sv])
            av = jnp.where(av > 0, av, av * 0.2)
            wv = jnp.where(sv == dv, 0.0, jnp.exp(av))
            for l in range(16):
                wsp = jnp.take_along_axis(
                    wv, jnp.full((16,), l, jnp.int32), axis=0)
                rr = j0 + l
                for cc in range(D // 16):
                    rows_v[rr, pl.ds(cc * 16, 16)] = (
                        rows_v[rr, pl.ds(cc * 16, 16)] * wsp)
            ks, vs = plsc.sort_key_val(dv, wv)
            cs = plsc.cumsum(vs)
            jsplat = jnp.full((16,), j, jnp.int32)
            ktmp_v[j, pl.ds(0, 16)] = ks
            ctmp_v[j, pl.ds(0, 16)] = cs
            kprev = plsc.load_gather(ktmp_v, [jsplat, jnp.maximum(lanes - 1, 0)])
            knext = plsc.load_gather(ktmp_v, [jsplat, jnp.minimum(lanes + 1, 15)])
            start = jnp.logical_or(lanes == 0, ks != kprev)
            end = jnp.logical_or(lanes == 15, ks != knext)
            st = plsc.cummax(jnp.where(start, lanes, 0))
            cprev = plsc.load_gather(ctmp_v, [jsplat, jnp.maximum(st - 1, 0)])
            cprev = jnp.where(st == 0, 0.0, cprev)
            plsc.addupdate_scatter(
                accw_v, [jnp.right_shift(ks, 7), jnp.bitwise_and(ks, 127)],
                cs - cprev, mask=end)

    def scatter(b, off, n, sem):
        pltpu.async_copy(rows_v.at[pl.ds(off, n)],
                         accf_sh.at[ibufs[b].at[1, pl.ds(off, n)]],
                         sem, add=True)

    def scatter_wait(b, off, n, sem):
        pltpu.make_async_copy(rows_v.at[pl.ds(off, n)],
                              accf_sh.at[ibufs[b].at[1, pl.ds(off, n)]],
                              sem).wait()

    def run_batch2(g, b, prefetch):
        _drain_s2(b)
        if prefetch:
            load_idx(g + 1, b ^ 1)
        gather(b, H0, H1, semB)
        gather_wait(b, 0, H0, semA)
        compute(b, 0, H0)
        scatter(b, 0, H0, semS)
        gather_wait(b, H0, H1, semB)
        compute(b, H0, H1)
        scatter_wait(b, 0, H0, semS)
        scatter(b, H0, H1, semS2)
        if prefetch:
            load_idx_wait(g + 1, b ^ 1)
            gather(b ^ 1, 0, H0, semA)

    load_idx(0, 0)
    load_idx_wait(0, 0)
    gather(0, 0, H0, semA)

    def pairbody(i, carry):
        run_batch2(2 * i, 0, True)
        run_batch2(2 * i + 1, 1, True)
        return carry

    lax.fori_loop(0, NB // 2, pairbody, 0)
    run_batch2(NB - 1, 0, False)
    _drain_s2(0)

    pltpu.sync_copy(accw_v, accw_sh.at[wid_v], add=True)
    plsc.subcore_barrier()

    pltpu.sync_copy(accf_sh.at[pl.ds(s * RPT, RPT)],
                    parts_hbm.at[c, pl.ds(s * RPT, RPT)])

    @pl.when(s < WR // 8)
    def _():
        pltpu.sync_copy(accw_sh.at[pl.ds(s * 8, 8)],
                        partsw_hbm.at[c, pl.ds(s * 8, 8)])


def _comb_bn_body(parts_ref, pw_ref, xl_ref, sa_ref, bias_ref,
                  gamma_ref, beta_ref, out_ref):
    accf = parts_ref[0, :N] + parts_ref[1, :N]
    accw = pw_ref[0, :N] + pw_ref[1, :N]
    als = sa_ref[...]
    als = jnp.where(als > 0, als, als * 0.2)
    ws = jnp.exp(als)
    num = accf + ws * xl_ref[...]
    den = accw + ws
    o = num / den + bias_ref[...]
    mu = jnp.mean(o, axis=0, keepdims=True)
    xc = o - mu
    var = jnp.mean(xc * xc, axis=0, keepdims=True)
    y = xc * lax.rsqrt(var + 1e-5) * gamma_ref[...] + beta_ref[...]
    out_ref[...] = jnp.maximum(y, 0.0)


_comb_bn = pl.pallas_call(
    _comb_bn_body,
    out_shape=jax.ShapeDtypeStruct((N, D), jnp.float32),
)


def kernel(x, edge_index, embedding, W, att_i, att_j, att_em_i, att_em_j,
           bias, gamma, beta):
    vi = att_i.reshape(1, D)
    vj = att_j.reshape(1, D)
    vei = att_em_i.reshape(1, D)
    vej = att_em_j.reshape(1, D)
    xl, aij = _proj(x, embedding, W, vi, vj, vei, vej)
    eib = edge_index.reshape(2, TB, K).transpose(1, 0, 2)
    parts, partsw = _edge_kernel(eib, xl, aij)
    pw = partsw.reshape(NC, NP, 1)
    sa_col = (aij[0] + aij[1])[:, None]
    return _comb_bn(parts, pw, xl, sa_col, bias.reshape(1, D),
                    gamma.reshape(1, D), beta.reshape(1, D))
